# Initial kernel scaffold; baseline (speedup 1.0000x reference)
import jax
import jax.numpy as jnp
from jax.experimental import pallas as pl


def kernel(x, edge_index, conv_w, conv_b, gamma, beta):
    raise NotImplementedError("write your pallas kernel here")


# speedup vs baseline: 8154.2171x; 8154.2171x over previous
import functools

import jax
import jax.numpy as jnp
from jax import lax
from jax.experimental import pallas as pl
from jax.experimental.pallas import tpu as pltpu
from jax.experimental.pallas import tpu_sc as plsc

B, C, N, K = 2, 128, 10000, 32
OUT = 128
GROUPS = 4
NT = B * N

_NC, _NS = 2, 16
_NW = _NC * _NS
_CHUNK = 4
_NODES_PER_W = 628
_NTPAD = _NODES_PER_W * _NW
_NCHUNKS = _NODES_PER_W // _CHUNK

_RB = 2000
_NBLK = NT // _RB
_NPB = N // _RB


def _sc_gather_max(xt, src, dst):
    mesh = plsc.VectorSubcoreMesh(core_axis_name="c", subcore_axis_name="s")

    @functools.partial(
        pl.kernel,
        mesh=mesh,
        out_type=jax.ShapeDtypeStruct((_NTPAD, C), jnp.float32),
        scratch_types=[
            pltpu.VMEM((_CHUNK * K,), jnp.int32),
            pltpu.VMEM((_CHUNK * K,), jnp.int32),
            pltpu.VMEM((_CHUNK * K, C), jnp.float32),
            pltpu.VMEM((_CHUNK * K, C), jnp.float32),
            pltpu.VMEM((_CHUNK, C), jnp.float32),
            pltpu.SemaphoreType.DMA,
            pltpu.SemaphoreType.DMA,
        ],
    )
    def body(xt_hbm, src_hbm, dst_hbm, out_hbm,
             sidx, didx, srows, drows, obuf, sem_s, sem_d):
        wid = lax.axis_index("s") * _NC + lax.axis_index("c")
        node0 = wid * _NODES_PER_W

        def chunk_body(ci, carry):
            nbase = node0 + ci * _CHUNK
            ibase = nbase * K
            pltpu.sync_copy(src_hbm.at[pl.ds(ibase, _CHUNK * K)], sidx)
            pltpu.sync_copy(dst_hbm.at[pl.ds(ibase, _CHUNK * K)], didx)
            cp_s = pltpu.async_copy(xt_hbm.at[sidx], srows, sem_s)
            cp_d = pltpu.async_copy(xt_hbm.at[didx], drows, sem_d)
            cp_s.wait()
            cp_d.wait()
            for nd in range(_CHUNK):
                def kbody(k, accs):
                    r = nd * K + k
                    return tuple(
                        jnp.maximum(
                            accs[cc],
                            srows[r, pl.ds(cc * 16, 16)]
                            - drows[r, pl.ds(cc * 16, 16)])
                        for cc in range(8))
                accs = tuple(
                    jnp.full((16,), -jnp.inf, jnp.float32) for _ in range(8))
                accs = lax.fori_loop(0, K, kbody, accs, unroll=4)
                for cc in range(8):
                    obuf[nd, pl.ds(cc * 16, 16)] = accs[cc]
            pltpu.sync_copy(obuf, out_hbm.at[pl.ds(nbase, _CHUNK)])
            return carry

        lax.fori_loop(0, _NCHUNKS, chunk_body, 0)

    return body(xt, src, dst)


def _conv_stats_body(xt_b, h2_b, w1, w2, bias, y_b, ssum, ssq):
    i = pl.program_id(0)
    y = (jnp.dot(xt_b[...], w1[...], preferred_element_type=jnp.float32)
         + jnp.dot(h2_b[...], w2[...], preferred_element_type=jnp.float32)
         + bias[...])
    y_b[...] = y

    @pl.when(i == 0)
    def _init():
        ssum[...] = jnp.zeros_like(ssum)
        ssq[...] = jnp.zeros_like(ssq)

    ssum[0, :] += jnp.sum(y, axis=0)
    ssq[0, :] += jnp.sum(y * y, axis=0)


def _bn_gelu_body(y_b, ssum, ssq, gamma, beta, out_b):
    mean = ssum[0, :] * (1.0 / NT)
    var = ssq[0, :] * (1.0 / NT) - mean * mean
    rstd = lax.rsqrt(var + 1e-5)
    g = gamma[0, :] * rstd
    bt = beta[0, :] - mean * g
    yn = y_b[...] * g[None, :] + bt[None, :]
    o = 0.5 * yn * (1.0 + lax.erf(yn * 0.7071067811865476))
    out_b[...] = o.T[None]


def kernel(x, edge_index, conv_w, conv_b, gamma, beta):
    xt = jnp.transpose(x[..., 0], (0, 2, 1)).reshape(NT, C)

    offs = (jnp.arange(B, dtype=jnp.int32) * N)[:, None, None]
    src = jnp.pad((edge_index[0] + offs).reshape(NT * K),
                  (0, (_NTPAD - NT) * K))
    dst = jnp.pad((edge_index[1] + offs).reshape(NT * K),
                  (0, (_NTPAD - NT) * K))

    h2 = _sc_gather_max(xt, src, dst)

    cin_g = 2 * C // GROUPS
    wg = conv_w[:, :, 0, 0].reshape(GROUPS, OUT // GROUPS, cin_g)
    wbd = jnp.zeros((2 * C, OUT), jnp.float32)
    for g in range(GROUPS):
        wbd = wbd.at[g * cin_g:(g + 1) * cin_g,
                     g * (OUT // GROUPS):(g + 1) * (OUT // GROUPS)].set(
                         jnp.transpose(wg[g]))
    w1, w2 = wbd[:C], wbd[C:]

    y, ssum, ssq = pl.pallas_call(
        _conv_stats_body,
        grid=(_NBLK,),
        in_specs=[
            pl.BlockSpec((_RB, C), lambda i: (i, 0)),
            pl.BlockSpec((_RB, C), lambda i: (i, 0)),
            pl.BlockSpec((C, OUT), lambda i: (0, 0)),
            pl.BlockSpec((C, OUT), lambda i: (0, 0)),
            pl.BlockSpec((1, OUT), lambda i: (0, 0)),
        ],
        out_specs=[
            pl.BlockSpec((_RB, OUT), lambda i: (i, 0)),
            pl.BlockSpec((8, OUT), lambda i: (0, 0)),
            pl.BlockSpec((8, OUT), lambda i: (0, 0)),
        ],
        out_shape=[
            jax.ShapeDtypeStruct((NT, OUT), jnp.float32),
            jax.ShapeDtypeStruct((8, OUT), jnp.float32),
            jax.ShapeDtypeStruct((8, OUT), jnp.float32),
        ],
    )(xt, h2, w1, w2, conv_b.reshape(1, OUT))

    out3 = pl.pallas_call(
        _bn_gelu_body,
        grid=(B,),
        in_specs=[
            pl.BlockSpec((N, OUT), lambda i: (i, 0)),
            pl.BlockSpec((8, OUT), lambda i: (0, 0)),
            pl.BlockSpec((8, OUT), lambda i: (0, 0)),
            pl.BlockSpec((1, OUT), lambda i: (0, 0)),
            pl.BlockSpec((1, OUT), lambda i: (0, 0)),
        ],
        out_specs=pl.BlockSpec((1, OUT, N), lambda i: (i, 0, 0)),
        out_shape=jax.ShapeDtypeStruct((B, OUT, N), jnp.float32),
    )(y, ssum, ssq, gamma.reshape(1, OUT), beta.reshape(1, OUT))

    return out3[..., None]
